# baseline (speedup 1.0000x reference)
import functools

import jax
import jax.numpy as jnp
from jax.experimental import pallas as pl
from jax.experimental.pallas import tpu as pltpu


def _scse_kernel(x_ref, w1_ref, b1_ref, w2_ref, b2_ref, ws_ref, bs_ref,
                 o_ref, *, inv_hw):
    xv = x_ref[0]

    pooled = jnp.sum(xv, axis=(1, 2), dtype=jnp.float32)[None, :] * inv_hw
    h = jnp.maximum(
        jnp.dot(pooled, w1_ref[...],
                preferred_element_type=jnp.float32) + b1_ref[...],
        0.0,
    )
    cg_row = jax.nn.sigmoid(
        jnp.dot(h, w2_ref[...],
                preferred_element_type=jnp.float32) + b2_ref[...]
    )
    cgate = cg_row.reshape(cg_row.shape[1], 1, 1)

    ws_col = ws_ref[...][:, :, None]
    slogit = jnp.sum(xv * ws_col, axis=0, keepdims=True,
                     dtype=jnp.float32) + bs_ref[0]
    sgate = jax.nn.sigmoid(slogit)

    o_ref[0] = xv * (cgate + sgate)


def kernel(x_nchw, w1, b1, w2, b2, ws, bs):
    N, C, H, W = x_nchw.shape
    Cr = w1.shape[1]

    body = functools.partial(_scse_kernel, inv_hw=1.0 / float(H * W))

    return pl.pallas_call(
        body,
        out_shape=jax.ShapeDtypeStruct((N, C, H, W), x_nchw.dtype),
        grid=(N,),
        in_specs=[
            pl.BlockSpec((1, C, H, W), lambda n: (n, 0, 0, 0)),
            pl.BlockSpec((C, Cr), lambda n: (0, 0)),
            pl.BlockSpec((1, Cr), lambda n: (0, 0)),
            pl.BlockSpec((Cr, C), lambda n: (0, 0)),
            pl.BlockSpec((1, C), lambda n: (0, 0)),
            pl.BlockSpec((C, 1), lambda n: (0, 0)),
            pl.BlockSpec(memory_space=pltpu.SMEM),
        ],
        out_specs=pl.BlockSpec((1, C, H, W), lambda n: (n, 0, 0, 0)),
        compiler_params=pltpu.CompilerParams(
            dimension_semantics=("parallel",),
            vmem_limit_bytes=56 * 1024 * 1024,
        ),
    )(x_nchw, w1, b1, w2, b2, ws, bs)


# speedup vs baseline: 1.7487x; 1.7487x over previous
import functools

import jax
import jax.numpy as jnp
from jax.experimental import pallas as pl
from jax.experimental.pallas import tpu as pltpu


def _scse_kernel(x_ref, w1_ref, b1_ref, w2_ref, b2_ref, ws_ref, bs_ref,
                 o_ref, *, inv_hw):
    xv = x_ref[0]

    pooled = jnp.sum(xv, axis=(1, 2), dtype=jnp.float32)[None, :] * inv_hw
    h = jnp.maximum(
        jnp.dot(pooled, w1_ref[...],
                preferred_element_type=jnp.float32) + b1_ref[...],
        0.0,
    )
    cg_row = jax.nn.sigmoid(
        jnp.dot(h, w2_ref[...],
                preferred_element_type=jnp.float32) + b2_ref[...]
    )
    cgate = cg_row.reshape(cg_row.shape[1], 1, 1)

    ws_col = ws_ref[...][:, :, None]
    slogit = jnp.sum(xv * ws_col, axis=0, keepdims=True,
                     dtype=jnp.float32) + bs_ref[0]
    sgate = jax.nn.sigmoid(slogit)

    o_ref[0] = xv * (cgate + sgate)


def kernel(x_nchw, w1, b1, w2, b2, ws, bs):
    N, C, H, W = x_nchw.shape
    Cr = w1.shape[1]
    Wv = 128
    Hv = (H * W) // Wv
    xv4 = x_nchw.reshape(N, C, Hv, Wv)

    body = functools.partial(_scse_kernel, inv_hw=1.0 / float(H * W))

    out = pl.pallas_call(
        body,
        out_shape=jax.ShapeDtypeStruct((N, C, Hv, Wv), x_nchw.dtype),
        grid=(N,),
        in_specs=[
            pl.BlockSpec((1, C, Hv, Wv), lambda n: (n, 0, 0, 0)),
            pl.BlockSpec((C, Cr), lambda n: (0, 0)),
            pl.BlockSpec((1, Cr), lambda n: (0, 0)),
            pl.BlockSpec((Cr, C), lambda n: (0, 0)),
            pl.BlockSpec((1, C), lambda n: (0, 0)),
            pl.BlockSpec((C, 1), lambda n: (0, 0)),
            pl.BlockSpec(memory_space=pltpu.SMEM),
        ],
        out_specs=pl.BlockSpec((1, C, Hv, Wv), lambda n: (n, 0, 0, 0)),
        compiler_params=pltpu.CompilerParams(
            dimension_semantics=("parallel",),
            vmem_limit_bytes=56 * 1024 * 1024,
        ),
    )(xv4, w1, b1, w2, b2, ws, bs)
    return out.reshape(N, C, H, W)


# speedup vs baseline: 6.0084x; 3.4359x over previous
import functools

import jax
import jax.numpy as jnp
from jax.experimental import pallas as pl
from jax.experimental.pallas import tpu as pltpu


def _scse_kernel(x_ref, w1_ref, b1_ref, w2_ref, b2_ref, ws_ref, bs_ref,
                 o_ref, *, inv_hw):
    xv = x_ref[0]

    pooled = jnp.sum(xv, axis=0, keepdims=True,
                     dtype=jnp.float32) * inv_hw
    h = jnp.maximum(
        jnp.dot(pooled, w1_ref[...],
                preferred_element_type=jnp.float32) + b1_ref[...],
        0.0,
    )
    cgate = jax.nn.sigmoid(
        jnp.dot(h, w2_ref[...],
                preferred_element_type=jnp.float32) + b2_ref[...]
    )

    slogit = jnp.dot(xv, ws_ref[...],
                     preferred_element_type=jnp.float32) + bs_ref[0]
    sgate = jax.nn.sigmoid(slogit)

    o_ref[0] = xv * (cgate + sgate)


def kernel(x_nchw, w1, b1, w2, b2, ws, bs):
    N, C, H, W = x_nchw.shape
    HW = H * W
    Cr = w1.shape[1]

    xt = jnp.transpose(x_nchw, (0, 2, 3, 1)).reshape(N, HW, C)

    body = functools.partial(_scse_kernel, inv_hw=1.0 / float(HW))

    out = pl.pallas_call(
        body,
        out_shape=jax.ShapeDtypeStruct((N, HW, C), x_nchw.dtype),
        grid=(N,),
        in_specs=[
            pl.BlockSpec((1, HW, C), lambda n: (n, 0, 0)),
            pl.BlockSpec((C, Cr), lambda n: (0, 0)),
            pl.BlockSpec((1, Cr), lambda n: (0, 0)),
            pl.BlockSpec((Cr, C), lambda n: (0, 0)),
            pl.BlockSpec((1, C), lambda n: (0, 0)),
            pl.BlockSpec((C, 1), lambda n: (0, 0)),
            pl.BlockSpec(memory_space=pltpu.SMEM),
        ],
        out_specs=pl.BlockSpec((1, HW, C), lambda n: (n, 0, 0)),
        compiler_params=pltpu.CompilerParams(
            dimension_semantics=("parallel",),
            vmem_limit_bytes=56 * 1024 * 1024,
        ),
    )(xt, w1, b1, w2, b2, ws, bs)
    return jnp.transpose(out.reshape(N, H, W, C), (0, 3, 1, 2))


# speedup vs baseline: 6.4680x; 1.0765x over previous
import functools

import jax
import jax.numpy as jnp
from jax.experimental import pallas as pl
from jax.experimental.pallas import tpu as pltpu


def _scse_kernel(x_ref, w1t_ref, b1_ref, w2_ref, b2_ref, wst_ref, bs_ref,
                 o_ref, *, inv_hw):
    xv = x_ref[0]

    pooled = jnp.sum(xv, axis=0, keepdims=True,
                     dtype=jnp.float32) * inv_hw
    h = jnp.maximum(
        jax.lax.dot_general(pooled, w1t_ref[...],
                            (((1,), (1,)), ((), ())),
                            preferred_element_type=jnp.float32) + b1_ref[...],
        0.0,
    )
    cgate = jax.nn.sigmoid(
        jnp.dot(h, w2_ref[...],
                preferred_element_type=jnp.float32) + b2_ref[...]
    )

    slogit = jnp.sum(xv * wst_ref[...], axis=1, keepdims=True,
                     dtype=jnp.float32) + bs_ref[0]
    sgate = jax.nn.sigmoid(slogit)

    o_ref[0] = xv * (cgate + sgate)


def kernel(x_nchw, w1, b1, w2, b2, ws, bs):
    N, C, H, W = x_nchw.shape
    HW = H * W
    Cr = w1.shape[1]

    xt = jnp.transpose(x_nchw, (0, 2, 3, 1)).reshape(N, HW, C)
    w1t = w1.T
    wst = ws.reshape(1, C)

    body = functools.partial(_scse_kernel, inv_hw=1.0 / float(HW))

    out = pl.pallas_call(
        body,
        out_shape=jax.ShapeDtypeStruct((N, HW, C), x_nchw.dtype),
        grid=(N,),
        in_specs=[
            pl.BlockSpec((1, HW, C), lambda n: (n, 0, 0)),
            pl.BlockSpec((Cr, C), lambda n: (0, 0)),
            pl.BlockSpec((1, Cr), lambda n: (0, 0)),
            pl.BlockSpec((Cr, C), lambda n: (0, 0)),
            pl.BlockSpec((1, C), lambda n: (0, 0)),
            pl.BlockSpec((1, C), lambda n: (0, 0)),
            pl.BlockSpec(memory_space=pltpu.SMEM),
        ],
        out_specs=pl.BlockSpec((1, HW, C), lambda n: (n, 0, 0)),
        compiler_params=pltpu.CompilerParams(
            dimension_semantics=("parallel",),
            vmem_limit_bytes=56 * 1024 * 1024,
        ),
    )(xt, w1t, b1, w2, b2, wst, bs)
    return jnp.transpose(out.reshape(N, H, W, C), (0, 3, 1, 2))


# speedup vs baseline: 6.7344x; 1.0412x over previous
import functools

import jax
import jax.numpy as jnp
from jax.experimental import pallas as pl
from jax.experimental.pallas import tpu as pltpu


def _scse_kernel(x_ref, w1t_ref, b1_ref, w2_ref, b2_ref, wst_ref, bs_ref,
                 o_ref, *, inv_hw):
  for i in range(x_ref.shape[0]):
    xv = x_ref[i]

    pooled = jnp.sum(xv, axis=0, keepdims=True,
                     dtype=jnp.float32) * inv_hw
    h = jnp.maximum(
        jax.lax.dot_general(pooled, w1t_ref[...],
                            (((1,), (1,)), ((), ())),
                            preferred_element_type=jnp.float32) + b1_ref[...],
        0.0,
    )
    cgate = jax.nn.sigmoid(
        jnp.dot(h, w2_ref[...],
                preferred_element_type=jnp.float32) + b2_ref[...]
    )

    slogit = jnp.sum(xv * wst_ref[...], axis=1, keepdims=True,
                     dtype=jnp.float32) + bs_ref[0]
    sgate = jax.nn.sigmoid(slogit)

    o_ref[i] = xv * (cgate + sgate)


def kernel(x_nchw, w1, b1, w2, b2, ws, bs):
    N, C, H, W = x_nchw.shape
    HW = H * W
    Cr = w1.shape[1]

    xt = jnp.transpose(x_nchw, (0, 2, 3, 1)).reshape(N, HW, C)
    w1t = w1.T
    wst = ws.reshape(1, C)

    body = functools.partial(_scse_kernel, inv_hw=1.0 / float(HW))

    out = pl.pallas_call(
        body,
        out_shape=jax.ShapeDtypeStruct((N, HW, C), x_nchw.dtype),
        grid=(N // 2,),
        in_specs=[
            pl.BlockSpec((2, HW, C), lambda n: (n, 0, 0)),
            pl.BlockSpec((Cr, C), lambda n: (0, 0)),
            pl.BlockSpec((1, Cr), lambda n: (0, 0)),
            pl.BlockSpec((Cr, C), lambda n: (0, 0)),
            pl.BlockSpec((1, C), lambda n: (0, 0)),
            pl.BlockSpec((1, C), lambda n: (0, 0)),
            pl.BlockSpec(memory_space=pltpu.SMEM),
        ],
        out_specs=pl.BlockSpec((2, HW, C), lambda n: (n, 0, 0)),
        compiler_params=pltpu.CompilerParams(
            dimension_semantics=("parallel",),
            vmem_limit_bytes=56 * 1024 * 1024,
        ),
    )(xt, w1t, b1, w2, b2, wst, bs)
    return jnp.transpose(out.reshape(N, H, W, C), (0, 3, 1, 2))
